# baseline (speedup 1.0000x reference)
import functools
import jax
import jax.numpy as jnp
from jax import lax
from jax.experimental import pallas as pl
from jax.experimental.pallas import tpu as pltpu
from jax.experimental.pallas import tpu_sc as plsc

_NW = 32
_UNROLL = 4


def _make_sc(B, S, D, R):
    rows_per_worker = S // _NW
    n_chunks = rows_per_worker // R
    CHUNK = R * D
    mesh = plsc.VectorSubcoreMesh(core_axis_name="c", subcore_axis_name="s")

    @functools.partial(
        pl.kernel,
        out_type=jax.ShapeDtypeStruct((B * S * D,), jnp.float32),
        mesh=mesh,
        scratch_types=[
            pltpu.VMEM((CHUNK,), jnp.float32),
            pltpu.VMEM((CHUNK,), jnp.float32),
        ],
    )
    def k(x_hbm, pe_hbm, out_hbm, pe_v, x_v):
        wid = lax.axis_index("s") * 2 + lax.axis_index("c")
        pe_base = wid * rows_per_worker * D

        def chunk_body(c, carry):
            pe_off = pe_base + c * CHUNK
            pltpu.sync_copy(pe_hbm.at[pl.ds(pe_off, CHUNK)], pe_v)

            def batch_body(b, carry2):
                x_off = b * (S * D) + pe_off
                pltpu.sync_copy(x_hbm.at[pl.ds(x_off, CHUNK)], x_v)

                def add_body(i, carry3):
                    base = i * (16 * _UNROLL)
                    for u in range(_UNROLL):
                        sl = pl.ds(base + u * 16, 16)
                        x_v[sl] = x_v[sl] + pe_v[sl]
                    return carry3

                lax.fori_loop(0, CHUNK // (16 * _UNROLL), add_body, 0)
                pltpu.sync_copy(x_v, out_hbm.at[pl.ds(x_off, CHUNK)])
                return carry2

            lax.fori_loop(0, B, batch_body, 0)
            return carry

        lax.fori_loop(0, n_chunks, chunk_body, 0)

    return k


def kernel(x, pe_weight):
    B, S, D = x.shape
    k = _make_sc(B, S, D, 32)
    out = k(x.reshape(-1), pe_weight.reshape(-1))
    return out.reshape(B, S, D)


# speedup vs baseline: 5.6653x; 5.6653x over previous
import jax
import jax.numpy as jnp
from jax.experimental import pallas as pl


def _add_pe_kernel(x_ref, pe_ref, o_ref):
    i = pl.program_id(0)
    bs = x_ref.shape[1]
    o_ref[...] = x_ref[...] + pe_ref[pl.ds(i * bs, bs), :][None, :, :]


def kernel(x, pe_weight):
    B, S, D = x.shape
    BS = 512
    grid = (S // BS,)
    return pl.pallas_call(
        _add_pe_kernel,
        grid=grid,
        in_specs=[
            pl.BlockSpec((B, BS, D), lambda i: (0, i, 0)),
            pl.BlockSpec((S, D), lambda i: (0, 0)),
        ],
        out_specs=pl.BlockSpec((B, BS, D), lambda i: (0, i, 0)),
        out_shape=jax.ShapeDtypeStruct((B, S, D), x.dtype),
    )(x, pe_weight)
